# baseline (speedup 1.0000x reference)
import functools

import jax
import jax.numpy as jnp
from jax import lax
from jax.experimental import pallas as pl
from jax.experimental.pallas import tpu as pltpu
from jax.experimental.pallas import tpu_sc as plsc

CHUNK = 128
NUM_TILES = 16
NUM_CORES = 2


def _sc_aggregate(xp3, srcp, dstp, cfg, np_, slots, n_pairs_max):
    half = np_ // 2
    init_rows = half // NUM_TILES
    mesh = plsc.VectorSubcoreMesh(core_axis_name="c", subcore_axis_name="s")

    @functools.partial(
        pl.kernel,
        mesh=mesh,
        out_type=jax.ShapeDtypeStruct((np_, 2, 128), jnp.float32),
        scratch_types=[
            pltpu.VMEM_SHARED((half + 8, 2, 128), jnp.float32),
            pltpu.VMEM((16,), jnp.int32),
            pltpu.VMEM((CHUNK,), jnp.int32),
            pltpu.VMEM((CHUNK,), jnp.int32),
            pltpu.VMEM((CHUNK,), jnp.int32),
            pltpu.VMEM((CHUNK,), jnp.int32),
            pltpu.VMEM((CHUNK, 2, 128), jnp.float32),
            pltpu.SemaphoreType.DMA,
            pltpu.SemaphoreType.DMA,
        ],
    )
    def body(xp_hbm, src_hbm, dst_hbm, cfg_hbm, out_hbm,
             acc, cfgv, gidx0, gidx1, sidx0, sidx1, rows, semi, semg):
        c = lax.axis_index("c")
        s = lax.axis_index("s")

        r0 = s * init_rows
        init_cp = pltpu.async_copy(
            xp_hbm.at[pl.ds(c * half + r0, init_rows)],
            acc.at[pl.ds(r0, init_rows)],
            semg,
        )
        pltpu.sync_copy(cfg_hbm, cfgv)
        cv = cfgv[pl.ds(0, 16)]
        ept = jnp.where(c == 0, cv[0], cv[1])
        n_pairs = jnp.where(c == 0, cv[2], cv[3])
        ept = pl.multiple_of(ept, 2 * CHUNK)
        ebase = c * slots + s * ept

        def eoff(j):
            return pl.multiple_of(ebase + j * CHUNK, CHUNK)

        def idx_fire(j, gi, si):
            pltpu.async_copy(src_hbm.at[pl.ds(eoff(j), CHUNK)], gi, semi)
            pltpu.async_copy(dst_hbm.at[pl.ds(eoff(j), CHUNK)], si, semi)

        def idx_wait(j, gi, si):
            pltpu.make_async_copy(
                src_hbm.at[pl.ds(eoff(j), CHUNK)], gi, semi).wait()
            pltpu.make_async_copy(
                dst_hbm.at[pl.ds(eoff(j), CHUNK)], si, semi).wait()

        def gather(gi):
            return pltpu.async_copy(xp_hbm.at[gi], rows, semg)

        def scat(si):
            pltpu.sync_copy(rows, acc.at[si], add=True)

        init_cp.wait()
        plsc.subcore_barrier()

        @pl.when(n_pairs > 0)
        def _():
            idx_fire(0, gidx0, sidx0)

        def pair_body(jj, carry):
            @pl.when(jj < n_pairs)
            def _():
                j = 2 * jj
                idx_wait(j, gidx0, sidx0)
                g = gather(gidx0)
                idx_fire(j + 1, gidx1, sidx1)
                g.wait()
                scat(sidx0)
                idx_wait(j + 1, gidx1, sidx1)
                g = gather(gidx1)

                @pl.when(jj + 1 < n_pairs)
                def _():
                    idx_fire(j + 2, gidx0, sidx0)

                g.wait()
                scat(sidx1)

            return carry

        lax.fori_loop(0, n_pairs_max, pair_body, 0)
        plsc.subcore_barrier()

        pltpu.sync_copy(
            acc.at[pl.ds(r0, init_rows)],
            out_hbm.at[pl.ds(c * half + r0, init_rows)],
        )

    return body(xp3, srcp, dstp, cfg)


def _tc_matmul(x, a, w):
    n, d = x.shape
    bm = 1000
    grid = (n // bm,)

    def mm_body(x_ref, a_ref, w_ref, o_ref):
        xa = a_ref[...] + 0.5 * x_ref[...]
        o_ref[...] = jnp.dot(xa, w_ref[...],
                             preferred_element_type=jnp.float32)

    return pl.pallas_call(
        mm_body,
        grid=grid,
        in_specs=[
            pl.BlockSpec((bm, d), lambda i: (i, 0)),
            pl.BlockSpec((bm, d), lambda i: (i, 0)),
            pl.BlockSpec((d, d), lambda i: (0, 0)),
        ],
        out_specs=pl.BlockSpec((bm, d), lambda i: (i, 0)),
        out_shape=jax.ShapeDtypeStruct((n, d), jnp.float32),
    )(x, a, w)


def kernel(x, edge_index, W):
    n, d = x.shape
    e = edge_index.shape[1]
    src = edge_index[0].astype(jnp.int32)
    dst = edge_index[1].astype(jnp.int32)

    rstep = 2 * NUM_TILES * 8
    np_ = ((n + 1 + rstep - 1) // rstep) * rstep
    half = np_ // 2
    xp = jnp.concatenate([x, jnp.zeros((np_ - n, d), jnp.float32)])

    grp = NUM_TILES * 2 * CHUNK
    slots = ((e + grp - 1) // grp) * grp + NUM_TILES * 2 * CHUNK + 2 * CHUNK
    keep0 = dst < half
    ki = keep0.astype(jnp.int32)
    pos0 = lax.associative_scan(jnp.add, ki) - 1
    pos1 = lax.associative_scan(jnp.add, 1 - ki) - 1
    m0 = jnp.sum(ki)
    slot = jnp.where(keep0, pos0, slots + pos1)
    dst_local = jnp.where(keep0, dst, dst - half)
    srcp = jnp.zeros((2 * slots,), jnp.int32).at[slot].set(
        src, unique_indices=True, mode="drop")
    junk = half + jnp.arange(2 * slots, dtype=jnp.int32) % 8
    dstp = junk.at[slot].set(dst_local, unique_indices=True, mode="drop")

    pair_sz = NUM_TILES * 2 * CHUNK
    ept0 = jnp.maximum((m0 + pair_sz - 1) // pair_sz, 1) * (2 * CHUNK)
    m1 = e - m0
    ept1 = jnp.maximum((m1 + pair_sz - 1) // pair_sz, 1) * (2 * CHUNK)
    cfg = jnp.stack([ept0, ept1, ept0 // (2 * CHUNK), ept1 // (2 * CHUNK)]
                    + [m0] * 12).astype(jnp.int32)
    n_pairs_max = slots // pair_sz + 1

    acc = _sc_aggregate(xp.reshape(np_, 2, 128), srcp, dstp, cfg,
                        np_, slots, n_pairs_max)
    return _tc_matmul(x, acc.reshape(np_, d)[:n], W)


# speedup vs baseline: 5.1593x; 5.1593x over previous
import functools

import jax
import jax.numpy as jnp
from jax import lax
from jax.experimental import pallas as pl
from jax.experimental.pallas import tpu as pltpu
from jax.experimental.pallas import tpu_sc as plsc

DH = 128
CHUNK = 128
NUM_TILES = 16
NUM_CORES = 2
GRP = 8


def _sc_aggregate(xh, src0, src1, dstp, n_nodes, n_groups):
    rows_per_tile = n_nodes // NUM_TILES
    mesh = plsc.VectorSubcoreMesh(core_axis_name="c", subcore_axis_name="s")

    @functools.partial(
        pl.kernel,
        mesh=mesh,
        out_type=jax.ShapeDtypeStruct((2 * n_nodes, DH), jnp.float32),
        scratch_types=[
            pltpu.VMEM_SHARED((n_nodes, DH), jnp.float32),
            pltpu.VMEM((2, GRP, CHUNK), jnp.int32),
            pltpu.VMEM((2, GRP, CHUNK), jnp.int32),
            pltpu.VMEM((2, CHUNK, DH), jnp.float32),
            pltpu.SemaphoreType.DMA,
            pltpu.SemaphoreType.DMA,
            pltpu.SemaphoreType.DMA,
        ],
    )
    def body(xh_hbm, src0_hbm, src1_hbm, dst_hbm, out_hbm,
             acc, isrc, idst, rows, semi, semg, sems):
        c = lax.axis_index("c")
        s = lax.axis_index("s")

        def idx_fire(g, p):
            @pl.when(c == 0)
            def _():
                pltpu.async_copy(src0_hbm.at[s, pl.ds(g * GRP, GRP)],
                                 isrc.at[p], semi)

            @pl.when(c != 0)
            def _():
                pltpu.async_copy(src1_hbm.at[s, pl.ds(g * GRP, GRP)],
                                 isrc.at[p], semi)

            pltpu.async_copy(dst_hbm.at[s, pl.ds(g * GRP, GRP)],
                             idst.at[p], semi)

        def idx_wait(g, p):
            pltpu.make_async_copy(src0_hbm.at[s, pl.ds(g * GRP, GRP)],
                                  isrc.at[p], semi).wait()
            pltpu.make_async_copy(dst_hbm.at[s, pl.ds(g * GRP, GRP)],
                                  idst.at[p], semi).wait()

        def gather(p, b, r):
            return pltpu.async_copy(xh_hbm.at[isrc.at[p, b]], rows.at[r],
                                    semg)

        def scat(p, b, r):
            return pltpu.async_copy(rows.at[r], acc.at[idst.at[p, b]], sems,
                                    add=True)

        def scat_wait(p, b, r):
            pltpu.make_async_copy(rows.at[r], acc.at[idst.at[p, b]],
                                  sems).wait()

        def group(g, p):
            idx_wait(g, p)
            for b in range(GRP):
                r = b % 2
                if b >= 2:
                    scat_wait(p, b - 2, r)
                gather(p, b, r).wait()
                scat(p, b, r)
            scat_wait(p, GRP - 2, 0)
            scat_wait(p, GRP - 1, 1)

            @pl.when(g + 2 < n_groups)
            def _():
                idx_fire(g + 2, p)

        r0 = s * rows_per_tile
        init_cp = pltpu.async_copy(
            xh_hbm.at[pl.ds(c * n_nodes + r0, rows_per_tile)],
            acc.at[pl.ds(r0, rows_per_tile)],
            semg,
        )
        idx_fire(0, 0)
        idx_fire(1, 1)
        init_cp.wait()
        plsc.subcore_barrier()

        def pair_body(g2, carry):
            group(2 * g2, 0)
            group(2 * g2 + 1, 1)
            return carry

        lax.fori_loop(0, n_groups // 2, pair_body, 0)
        plsc.subcore_barrier()

        pltpu.sync_copy(
            acc.at[pl.ds(r0, rows_per_tile)],
            out_hbm.at[pl.ds(c * n_nodes + r0, rows_per_tile)],
        )

    return body(xh, src0, src1, dstp)


def _tc_matmul(x, a0, a1, w):
    n, d = x.shape
    bm = 1000
    grid = (n // bm,)

    def mm_body(x_ref, a0_ref, a1_ref, w_ref, o_ref):
        xb = x_ref[...]
        xa0 = a0_ref[...] + 0.5 * xb[:, :DH]
        xa1 = a1_ref[...] + 0.5 * xb[:, DH:]
        o_ref[...] = jnp.dot(
            xa0, w_ref[:DH, :], preferred_element_type=jnp.float32
        ) + jnp.dot(xa1, w_ref[DH:, :], preferred_element_type=jnp.float32)

    return pl.pallas_call(
        mm_body,
        grid=grid,
        in_specs=[
            pl.BlockSpec((bm, d), lambda i: (i, 0)),
            pl.BlockSpec((bm, DH), lambda i: (i, 0)),
            pl.BlockSpec((bm, DH), lambda i: (i, 0)),
            pl.BlockSpec((d, d), lambda i: (0, 0)),
        ],
        out_specs=pl.BlockSpec((bm, d), lambda i: (i, 0)),
        out_shape=jax.ShapeDtypeStruct((n, d), jnp.float32),
    )(x, a0, a1, w)


def kernel(x, edge_index, W):
    n, d = x.shape
    e = edge_index.shape[1]
    src = edge_index[0].astype(jnp.int32)
    dst = edge_index[1].astype(jnp.int32)

    rstep = NUM_TILES * 8
    np_ = ((n + rstep - 1) // rstep) * rstep + rstep
    xp = jnp.concatenate([x, jnp.zeros((np_ - n, d), jnp.float32)])
    xh = xp.reshape(np_, 2, DH).swapaxes(0, 1).reshape(2 * np_, DH)

    step = NUM_TILES * 2 * GRP * CHUNK
    e_pad = ((e + step - 1) // step) * step
    n_groups = e_pad // (NUM_TILES * GRP * CHUNK)
    n_chunks = n_groups * GRP
    pad = e_pad - e
    src0 = jnp.concatenate([src, jnp.zeros((pad,), jnp.int32)])
    src1 = src0 + np_
    dstp = jnp.concatenate([dst, jnp.full((pad,), n, jnp.int32)])
    shp = (NUM_TILES, n_chunks, CHUNK)

    acch = _sc_aggregate(xh, src0.reshape(shp), src1.reshape(shp),
                         dstp.reshape(shp), np_, n_groups)
    return _tc_matmul(x, acch[:n], acch[np_:np_ + n], W)


# speedup vs baseline: 5.4098x; 1.0486x over previous
import functools

import jax
import jax.numpy as jnp
from jax import lax
from jax.experimental import pallas as pl
from jax.experimental.pallas import tpu as pltpu
from jax.experimental.pallas import tpu_sc as plsc

DH = 128
CHUNK = 128
NUM_TILES = 16
NUM_CORES = 2
GRP = 8


def _sc_aggregate(xh, src0, src1, dstp, n_nodes, n_groups):
    rows_per_tile = n_nodes // NUM_TILES
    mesh = plsc.VectorSubcoreMesh(core_axis_name="c", subcore_axis_name="s")

    @functools.partial(
        pl.kernel,
        mesh=mesh,
        out_type=jax.ShapeDtypeStruct((2 * n_nodes, DH), jnp.float32),
        scratch_types=[
            pltpu.VMEM_SHARED((n_nodes, DH), jnp.float32),
            pltpu.VMEM((2, GRP, CHUNK), jnp.int32),
            pltpu.VMEM((2, GRP, CHUNK), jnp.int32),
            pltpu.VMEM((2, CHUNK, DH), jnp.float32),
            pltpu.SemaphoreType.DMA,
            pltpu.SemaphoreType.DMA,
            pltpu.SemaphoreType.DMA,
        ],
    )
    def body(xh_hbm, src0_hbm, src1_hbm, dst_hbm, out_hbm,
             acc, isrc, idst, rows, semi, semg, sems):
        c = lax.axis_index("c")
        s = lax.axis_index("s")

        def idx_fire(g, p):
            @pl.when(c == 0)
            def _():
                pltpu.async_copy(src0_hbm.at[s, pl.ds(g * GRP, GRP)],
                                 isrc.at[p], semi)

            @pl.when(c != 0)
            def _():
                pltpu.async_copy(src1_hbm.at[s, pl.ds(g * GRP, GRP)],
                                 isrc.at[p], semi)

            pltpu.async_copy(dst_hbm.at[s, pl.ds(g * GRP, GRP)],
                             idst.at[p], semi)

        def idx_wait(g, p):
            pltpu.make_async_copy(src0_hbm.at[s, pl.ds(g * GRP, GRP)],
                                  isrc.at[p], semi).wait()
            pltpu.make_async_copy(dst_hbm.at[s, pl.ds(g * GRP, GRP)],
                                  idst.at[p], semi).wait()

        def gather(p, b, r):
            return pltpu.async_copy(xh_hbm.at[isrc.at[p, b]], rows.at[r],
                                    semg)

        def scat(p, b, r):
            return pltpu.async_copy(rows.at[r], acc.at[idst.at[p, b]], sems,
                                    add=True)

        def scat_wait(p, b, r):
            pltpu.make_async_copy(rows.at[r], acc.at[idst.at[p, b]],
                                  sems).wait()

        def gwait(p, b, r):
            pltpu.make_async_copy(xh_hbm.at[isrc.at[p, b]], rows.at[r],
                                  semg).wait()

        def group(g, p):
            idx_wait(g, p)
            gather(p, 0, 0)
            for b in range(GRP):
                r = b % 2
                if b + 1 < GRP:
                    if b >= 1:
                        scat_wait(p, b - 1, 1 - r)
                    gather(p, b + 1, 1 - r)
                gwait(p, b, r)
                scat(p, b, r)
            scat_wait(p, GRP - 2, 0)
            scat_wait(p, GRP - 1, 1)

            @pl.when(g + 2 < n_groups)
            def _():
                idx_fire(g + 2, p)

        r0 = s * rows_per_tile
        init_cp = pltpu.async_copy(
            xh_hbm.at[pl.ds(c * n_nodes + r0, rows_per_tile)],
            acc.at[pl.ds(r0, rows_per_tile)],
            semg,
        )
        idx_fire(0, 0)
        idx_fire(1, 1)
        init_cp.wait()
        plsc.subcore_barrier()

        def pair_body(g2, carry):
            group(2 * g2, 0)
            group(2 * g2 + 1, 1)
            return carry

        lax.fori_loop(0, n_groups // 2, pair_body, 0)
        plsc.subcore_barrier()

        pltpu.sync_copy(
            acc.at[pl.ds(r0, rows_per_tile)],
            out_hbm.at[pl.ds(c * n_nodes + r0, rows_per_tile)],
        )

    return body(xh, src0, src1, dstp)


def _tc_matmul(x, a0, a1, w):
    n, d = x.shape
    bm = 1000
    grid = (n // bm,)

    def mm_body(x_ref, a0_ref, a1_ref, w_ref, o_ref):
        xb = x_ref[...]
        xa0 = a0_ref[...] + 0.5 * xb[:, :DH]
        xa1 = a1_ref[...] + 0.5 * xb[:, DH:]
        o_ref[...] = jnp.dot(
            xa0, w_ref[:DH, :], preferred_element_type=jnp.float32
        ) + jnp.dot(xa1, w_ref[DH:, :], preferred_element_type=jnp.float32)

    return pl.pallas_call(
        mm_body,
        grid=grid,
        in_specs=[
            pl.BlockSpec((bm, d), lambda i: (i, 0)),
            pl.BlockSpec((bm, DH), lambda i: (i, 0)),
            pl.BlockSpec((bm, DH), lambda i: (i, 0)),
            pl.BlockSpec((d, d), lambda i: (0, 0)),
        ],
        out_specs=pl.BlockSpec((bm, d), lambda i: (i, 0)),
        out_shape=jax.ShapeDtypeStruct((n, d), jnp.float32),
    )(x, a0, a1, w)


def kernel(x, edge_index, W):
    n, d = x.shape
    e = edge_index.shape[1]
    src = edge_index[0].astype(jnp.int32)
    dst = edge_index[1].astype(jnp.int32)

    rstep = NUM_TILES * 8
    np_ = ((n + rstep - 1) // rstep) * rstep + rstep
    xp = jnp.concatenate([x, jnp.zeros((np_ - n, d), jnp.float32)])
    xh = xp.reshape(np_, 2, DH).swapaxes(0, 1).reshape(2 * np_, DH)

    step = NUM_TILES * 2 * GRP * CHUNK
    e_pad = ((e + step - 1) // step) * step
    n_groups = e_pad // (NUM_TILES * GRP * CHUNK)
    n_chunks = n_groups * GRP
    pad = e_pad - e
    src0 = jnp.concatenate([src, jnp.zeros((pad,), jnp.int32)])
    src1 = src0 + np_
    dstp = jnp.concatenate([dst, jnp.full((pad,), n, jnp.int32)])
    shp = (NUM_TILES, n_chunks, CHUNK)

    acch = _sc_aggregate(xh, src0.reshape(shp), src1.reshape(shp),
                         dstp.reshape(shp), np_, n_groups)
    return _tc_matmul(x, acch[:n], acch[np_:np_ + n], W)


# speedup vs baseline: 5.4214x; 1.0021x over previous
import functools

import jax
import jax.numpy as jnp
from jax import lax
from jax.experimental import pallas as pl
from jax.experimental.pallas import tpu as pltpu
from jax.experimental.pallas import tpu_sc as plsc

DH = 128
CHUNK = 128
NUM_TILES = 16
NUM_CORES = 2
GRP = 8


def _sc_aggregate(xh, src0, src1, dstp, n_nodes, n_groups):
    rows_per_tile = n_nodes // NUM_TILES
    mesh = plsc.VectorSubcoreMesh(core_axis_name="c", subcore_axis_name="s")

    @functools.partial(
        pl.kernel,
        mesh=mesh,
        out_type=jax.ShapeDtypeStruct((2 * n_nodes, DH), jnp.float32),
        scratch_types=[
            pltpu.VMEM_SHARED((n_nodes, DH), jnp.float32),
            pltpu.VMEM((2, GRP, CHUNK), jnp.int32),
            pltpu.VMEM((2, GRP, CHUNK), jnp.int32),
            pltpu.VMEM((2, CHUNK, DH), jnp.float32),
            pltpu.SemaphoreType.DMA,
            pltpu.SemaphoreType.DMA,
            pltpu.SemaphoreType.DMA,
        ],
    )
    def body(xh_hbm, src0_hbm, src1_hbm, dst_hbm, out_hbm,
             acc, isrc, idst, rows, semi, semg, sems):
        c = lax.axis_index("c")
        s = lax.axis_index("s")

        def idx_fire(g, p):
            @pl.when(c == 0)
            def _():
                pltpu.async_copy(src0_hbm.at[s, pl.ds(g * GRP, GRP)],
                                 isrc.at[p], semi)

            @pl.when(c != 0)
            def _():
                pltpu.async_copy(src1_hbm.at[s, pl.ds(g * GRP, GRP)],
                                 isrc.at[p], semi)

            pltpu.async_copy(dst_hbm.at[s, pl.ds(g * GRP, GRP)],
                             idst.at[p], semi)

        def idx_wait(g, p):
            pltpu.make_async_copy(src0_hbm.at[s, pl.ds(g * GRP, GRP)],
                                  isrc.at[p], semi).wait()
            pltpu.make_async_copy(dst_hbm.at[s, pl.ds(g * GRP, GRP)],
                                  idst.at[p], semi).wait()

        def gather(p, b, r):
            return pltpu.async_copy(xh_hbm.at[isrc.at[p, b]], rows.at[r],
                                    semg)

        def scat(p, b, r):
            return pltpu.async_copy(rows.at[r], acc.at[idst.at[p, b]], sems,
                                    add=True)

        def scat_wait(p, b, r):
            pltpu.make_async_copy(rows.at[r], acc.at[idst.at[p, b]],
                                  sems).wait()

        def gwait(p, b, r):
            pltpu.make_async_copy(xh_hbm.at[isrc.at[p, b]], rows.at[r],
                                  semg).wait()

        def group(g, p):
            idx_wait(g, p)
            gather(p, 0, 0)
            for b in range(GRP):
                r = b % 2
                if b + 1 < GRP:
                    if b >= 1:
                        scat_wait(p, b - 1, 1 - r)
                    gather(p, b + 1, 1 - r)
                gwait(p, b, r)
                scat(p, b, r)
            scat_wait(p, GRP - 2, 0)
            scat_wait(p, GRP - 1, 1)

            @pl.when(g + 2 < n_groups)
            def _():
                idx_fire(g + 2, p)

        r0 = s * rows_per_tile
        init_cp = pltpu.async_copy(
            xh_hbm.at[pl.ds(c * n_nodes + r0, rows_per_tile)],
            acc.at[pl.ds(r0, rows_per_tile)],
            semg,
        )
        idx_fire(0, 0)
        idx_fire(1, 1)
        init_cp.wait()
        plsc.subcore_barrier()

        def pair_body(g2, carry):
            group(2 * g2, 0)
            group(2 * g2 + 1, 1)
            return carry

        lax.fori_loop(0, n_groups // 2, pair_body, 0)
        plsc.subcore_barrier()

        pltpu.sync_copy(
            acc.at[pl.ds(r0, rows_per_tile)],
            out_hbm.at[pl.ds(c * n_nodes + r0, rows_per_tile)],
        )

    return body(xh, src0, src1, dstp)


def _tc_matmul(x, a0, a1, w):
    n, d = x.shape
    bm = 1000
    grid = (n // bm,)

    def mm_body(x_ref, a0_ref, a1_ref, w_ref, o_ref):
        xb = x_ref[...]
        xa0 = (a0_ref[...] + 0.5 * xb[:, :DH]).astype(jnp.bfloat16)
        xa1 = (a1_ref[...] + 0.5 * xb[:, DH:]).astype(jnp.bfloat16)
        wb = w_ref[...].astype(jnp.bfloat16)
        o_ref[...] = jnp.dot(
            xa0, wb[:DH, :], preferred_element_type=jnp.float32
        ) + jnp.dot(xa1, wb[DH:, :], preferred_element_type=jnp.float32)

    return pl.pallas_call(
        mm_body,
        grid=grid,
        in_specs=[
            pl.BlockSpec((bm, d), lambda i: (i, 0)),
            pl.BlockSpec((bm, DH), lambda i: (i, 0)),
            pl.BlockSpec((bm, DH), lambda i: (i, 0)),
            pl.BlockSpec((d, d), lambda i: (0, 0)),
        ],
        out_specs=pl.BlockSpec((bm, d), lambda i: (i, 0)),
        out_shape=jax.ShapeDtypeStruct((n, d), jnp.float32),
    )(x, a0, a1, w)


def kernel(x, edge_index, W):
    n, d = x.shape
    e = edge_index.shape[1]
    src = edge_index[0].astype(jnp.int32)
    dst = edge_index[1].astype(jnp.int32)

    rstep = NUM_TILES * 8
    np_ = ((n + rstep - 1) // rstep) * rstep + rstep
    xp = jnp.concatenate([x, jnp.zeros((np_ - n, d), jnp.float32)])
    xh = xp.reshape(np_, 2, DH).swapaxes(0, 1).reshape(2 * np_, DH)

    step = NUM_TILES * 2 * GRP * CHUNK
    e_pad = ((e + step - 1) // step) * step
    n_groups = e_pad // (NUM_TILES * GRP * CHUNK)
    n_chunks = n_groups * GRP
    pad = e_pad - e
    src0 = jnp.concatenate([src, jnp.zeros((pad,), jnp.int32)])
    src1 = src0 + np_
    dstp = jnp.concatenate([dst, jnp.full((pad,), n, jnp.int32)])
    shp = (NUM_TILES, n_chunks, CHUNK)

    acch = _sc_aggregate(xh, src0.reshape(shp), src1.reshape(shp),
                         dstp.reshape(shp), np_, n_groups)
    return _tc_matmul(x, acch[:n], acch[np_:np_ + n], W)
